# Initial kernel scaffold; baseline (speedup 1.0000x reference)
import jax
import jax.numpy as jnp
from jax.experimental import pallas as pl


def kernel(x, edge_index, W1, b1, W2, b2, Wl, bl):
    raise NotImplementedError("write your pallas kernel here")


# speedup vs baseline: 12.7615x; 12.7615x over previous
import functools

import jax
import jax.numpy as jnp
from jax import lax
from jax.experimental import pallas as pl
from jax.experimental.pallas import tpu as pltpu
from jax.experimental.pallas import tpu_sc as plsc

N = 10000
E = 320000
D = 128

NC = 2
NS = 16
NW = NC * NS
EPW = E // NW
CH = 80
NCH = EPW // CH
ACC_ROWS = 10240
RPS = ACC_ROWS // NS
ZR = 64
DEG_W = 16

BLK = 2000


_MESH = functools.partial(
    plsc.VectorSubcoreMesh, core_axis_name="c", subcore_axis_name="s"
)


def _sc_deg(dst):
    @functools.partial(
        pl.kernel,
        out_type=jax.ShapeDtypeStruct((NC, N, DEG_W), jnp.float32),
        mesh=_MESH(),
        scratch_types=[
            pltpu.VMEM((CH,), jnp.int32),
            pltpu.VMEM((CH, DEG_W), jnp.float32),
            pltpu.VMEM((ZR, DEG_W), jnp.float32),
            pltpu.VMEM_SHARED((ACC_ROWS, DEG_W), jnp.float32),
        ],
    )
    def deg_kernel(dst_hbm, out_hbm, dst_v, ones_v, zero_v, acc_sh):
        c = lax.axis_index("c")
        s = lax.axis_index("s")
        wid = c * NS + s

        @pl.loop(0, CH)
        def _(i):
            ones_v[i, :] = jnp.ones((DEG_W,), jnp.float32)

        @pl.loop(0, ZR)
        def _(i):
            zero_v[i, :] = jnp.zeros((DEG_W,), jnp.float32)

        @pl.loop(0, RPS // ZR)
        def _(k):
            pltpu.sync_copy(zero_v, acc_sh.at[pl.ds(s * RPS + k * ZR, ZR)])

        plsc.subcore_barrier()

        @pl.loop(0, NCH)
        def _(ch):
            base = wid * EPW + ch * CH
            pltpu.sync_copy(dst_hbm.at[pl.ds(base, CH)], dst_v)
            pltpu.sync_copy(ones_v, acc_sh.at[dst_v], add=True)

        plsc.subcore_barrier()

        @pl.when(s < NS - 1)
        def _():
            pltpu.sync_copy(
                acc_sh.at[pl.ds(s * RPS, RPS)],
                out_hbm.at[c, pl.ds(s * RPS, RPS)],
            )

        @pl.when(s == NS - 1)
        def _():
            pltpu.sync_copy(
                acc_sh.at[pl.ds((NS - 1) * RPS, N - (NS - 1) * RPS)],
                out_hbm.at[c, pl.ds((NS - 1) * RPS, N - (NS - 1) * RPS)],
            )

    return deg_kernel(dst)


def _sc_agg(hp, src, dst):
    @functools.partial(
        pl.kernel,
        out_type=jax.ShapeDtypeStruct((NC, N, D), jnp.float32),
        mesh=_MESH(),
        scratch_types=[
            pltpu.VMEM((CH,), jnp.int32),
            pltpu.VMEM((CH,), jnp.int32),
            pltpu.VMEM((CH, D), jnp.float32),
            pltpu.VMEM((ZR, D), jnp.float32),
            pltpu.VMEM_SHARED((ACC_ROWS, D), jnp.float32),
            pltpu.SemaphoreType.DMA,
        ],
    )
    def agg_kernel(hp_hbm, src_hbm, dst_hbm, out_hbm,
                   src_v, dst_v, rows_v, zero_v, acc_sh, sem):
        c = lax.axis_index("c")
        s = lax.axis_index("s")
        wid = c * NS + s

        @pl.loop(0, ZR)
        def _(i):
            for j in range(D // 16):
                zero_v[i, pl.ds(j * 16, 16)] = jnp.zeros((16,), jnp.float32)

        @pl.loop(0, RPS // ZR)
        def _(k):
            pltpu.sync_copy(zero_v, acc_sh.at[pl.ds(s * RPS + k * ZR, ZR)])

        plsc.subcore_barrier()

        @pl.loop(0, NCH)
        def _(ch):
            base = wid * EPW + ch * CH
            pltpu.sync_copy(src_hbm.at[pl.ds(base, CH)], src_v)
            pltpu.async_copy(hp_hbm.at[src_v], rows_v, sem).wait()
            pltpu.sync_copy(dst_hbm.at[pl.ds(base, CH)], dst_v)
            pltpu.sync_copy(rows_v, acc_sh.at[dst_v], add=True)

        plsc.subcore_barrier()

        @pl.when(s < NS - 1)
        def _():
            pltpu.sync_copy(
                acc_sh.at[pl.ds(s * RPS, RPS)],
                out_hbm.at[c, pl.ds(s * RPS, RPS)],
            )

        @pl.when(s == NS - 1)
        def _():
            pltpu.sync_copy(
                acc_sh.at[pl.ds((NS - 1) * RPS, N - (NS - 1) * RPS)],
                out_hbm.at[c, pl.ds((NS - 1) * RPS, N - (NS - 1) * RPS)],
            )

    return agg_kernel(hp, src, dst)


def _tc_matmul(x, W):
    def body(x_ref, w_ref, o_ref):
        o_ref[...] = jnp.dot(x_ref[...], w_ref[...],
                             preferred_element_type=jnp.float32)

    return pl.pallas_call(
        body,
        grid=(N // BLK,),
        in_specs=[
            pl.BlockSpec((BLK, D), lambda i: (i, 0)),
            pl.BlockSpec((D, D), lambda i: (0, 0)),
        ],
        out_specs=pl.BlockSpec((BLK, D), lambda i: (i, 0)),
        out_shape=jax.ShapeDtypeStruct((N, D), jnp.float32),
    )(x, W)


def _tc_prep(deg0, deg1, xW1):
    def body(d0_ref, d1_ref, xw_ref, dmat_ref, hp_ref):
        deg = d0_ref[:, 0:1] + d1_ref[:, 0:1] + 1.0
        dinv = lax.rsqrt(deg)
        dmat = jnp.broadcast_to(dinv, (BLK, D))
        dmat_ref[...] = dmat
        hp_ref[...] = xw_ref[...] * dmat

    return pl.pallas_call(
        body,
        grid=(N // BLK,),
        in_specs=[
            pl.BlockSpec((BLK, DEG_W), lambda i: (i, 0)),
            pl.BlockSpec((BLK, DEG_W), lambda i: (i, 0)),
            pl.BlockSpec((BLK, D), lambda i: (i, 0)),
        ],
        out_specs=[
            pl.BlockSpec((BLK, D), lambda i: (i, 0)),
            pl.BlockSpec((BLK, D), lambda i: (i, 0)),
        ],
        out_shape=[
            jax.ShapeDtypeStruct((N, D), jnp.float32),
            jax.ShapeDtypeStruct((N, D), jnp.float32),
        ],
    )(deg0, deg1, xW1)


def _tc_mid(a0, a1, hp, dmat, b2d, W):
    def body(a0_ref, a1_ref, hp_ref, dm_ref, b_ref, w_ref, o_ref):
        dm = dm_ref[...]
        z = dm * (a0_ref[...] + a1_ref[...] + hp_ref[...]) + b_ref[0:1, :]
        z = jnp.maximum(z, 0.0)
        o_ref[...] = jnp.dot(z, w_ref[...],
                             preferred_element_type=jnp.float32) * dm

    return pl.pallas_call(
        body,
        grid=(N // BLK,),
        in_specs=[
            pl.BlockSpec((BLK, D), lambda i: (i, 0)),
            pl.BlockSpec((BLK, D), lambda i: (i, 0)),
            pl.BlockSpec((BLK, D), lambda i: (i, 0)),
            pl.BlockSpec((BLK, D), lambda i: (i, 0)),
            pl.BlockSpec((8, D), lambda i: (0, 0)),
            pl.BlockSpec((D, D), lambda i: (0, 0)),
        ],
        out_specs=pl.BlockSpec((BLK, D), lambda i: (i, 0)),
        out_shape=jax.ShapeDtypeStruct((N, D), jnp.float32),
    )(a0, a1, hp, dmat, b2d, W)


def _tc_final(a0, a1, hp, dmat, b2d, Wlp, blp):
    def body(a0_ref, a1_ref, hp_ref, dm_ref, b_ref, w_ref, bl_ref, o_ref):
        z = dm_ref[...] * (a0_ref[...] + a1_ref[...] + hp_ref[...]) \
            + b_ref[0:1, :]
        z = jnp.maximum(z, 0.0)
        y = jnp.dot(z, w_ref[...], preferred_element_type=jnp.float32) \
            + bl_ref[0:1, :]
        o_ref[...] = jax.nn.sigmoid(y)

    return pl.pallas_call(
        body,
        grid=(N // BLK,),
        in_specs=[
            pl.BlockSpec((BLK, D), lambda i: (i, 0)),
            pl.BlockSpec((BLK, D), lambda i: (i, 0)),
            pl.BlockSpec((BLK, D), lambda i: (i, 0)),
            pl.BlockSpec((BLK, D), lambda i: (i, 0)),
            pl.BlockSpec((8, D), lambda i: (0, 0)),
            pl.BlockSpec((D, D), lambda i: (0, 0)),
            pl.BlockSpec((8, D), lambda i: (0, 0)),
        ],
        out_specs=pl.BlockSpec((BLK, D), lambda i: (i, 0)),
        out_shape=jax.ShapeDtypeStruct((N, D), jnp.float32),
    )(a0, a1, hp, dmat, b2d, Wlp, blp)


def kernel(x, edge_index, W1, b1, W2, b2, Wl, bl):
    ei = edge_index.astype(jnp.int32)
    src = ei[0]
    dst = ei[1]

    degp = _sc_deg(dst)
    xW1 = _tc_matmul(x, W1)
    dmat, hp1 = _tc_prep(degp[0], degp[1], xW1)

    a1 = _sc_agg(hp1, src, dst)
    b1_2d = jnp.broadcast_to(b1.reshape(1, D), (8, D))
    hp2 = _tc_mid(a1[0], a1[1], hp1, dmat, b1_2d, W2)

    a2 = _sc_agg(hp2, src, dst)
    b2_2d = jnp.broadcast_to(b2.reshape(1, D), (8, D))
    Wlp = jnp.pad(Wl, ((0, 0), (0, D - Wl.shape[1])))
    bl_2d = jnp.broadcast_to(bl.reshape(1, 1), (8, D))
    wide = _tc_final(a2[0], a2[1], hp2, dmat, b2_2d, Wlp, bl_2d)
    return wide[:, :1]
